# Initial kernel scaffold; baseline (speedup 1.0000x reference)
import jax
import jax.numpy as jnp
from jax.experimental import pallas as pl


def kernel(nodes, table, weight):
    raise NotImplementedError("write your pallas kernel here")


# speedup vs baseline: 5.8066x; 5.8066x over previous
import functools

import jax
import jax.numpy as jnp
from jax import lax
from jax.experimental import pallas as pl
from jax.experimental.pallas import tpu as pltpu
from jax.experimental.pallas import tpu_sc as plsc

NUM_NODES = 100000
EMBED_DIM = 256
NUM_CLASSES = 64
BATCH = 16384

NC = 2
NS = 16
NW = NC * NS
B_PER_W = BATCH // NW
CHUNK = 128
N_CHUNKS = B_PER_W // CHUNK

_MESH = plsc.VectorSubcoreMesh(core_axis_name="c", subcore_axis_name="s")


@functools.partial(
    pl.kernel,
    out_type=jax.ShapeDtypeStruct((BATCH, EMBED_DIM), jnp.float32),
    mesh=_MESH,
    scratch_types=[
        pltpu.VMEM((N_CHUNKS, CHUNK), jnp.int32),
        pltpu.VMEM((CHUNK, EMBED_DIM), jnp.float32),
        pltpu.VMEM((CHUNK, EMBED_DIM), jnp.float32),
        pltpu.SemaphoreType.DMA,
        pltpu.SemaphoreType.DMA,
    ],
)
def _sc_gather(table_hbm, idx_hbm, out_hbm, idx_v, rows_a, rows_b, sem_a, sem_b):
    wid = lax.axis_index("s") * NC + lax.axis_index("c")
    base = wid * B_PER_W
    pltpu.sync_copy(idx_hbm.at[wid], idx_v)
    rows = (rows_a, rows_b)
    sems = (sem_a, sem_b)
    cps = [pltpu.async_copy(table_hbm.at[idx_v.at[0]], rows_a, sem_a)]
    for c in range(N_CHUNKS):
        if c + 1 < N_CHUNKS:
            nxt = (c + 1) % 2
            cps.append(
                pltpu.async_copy(table_hbm.at[idx_v.at[c + 1]], rows[nxt], sems[nxt])
            )
        cps[c].wait()
        pltpu.sync_copy(rows[c % 2], out_hbm.at[pl.ds(base + c * CHUNK, CHUNK)])


def _mm_body(x_ref, w_ref, o_ref):
    o_ref[:] = lax.dot_general(
        x_ref[:], w_ref[:], (((1,), (1,)), ((), ())),
        preferred_element_type=jnp.float32,
    )


_BM = 2048


def _tc_matmul(gathered, weight):
    return pl.pallas_call(
        _mm_body,
        grid=(BATCH // _BM,),
        in_specs=[
            pl.BlockSpec((_BM, EMBED_DIM), lambda i: (i, 0)),
            pl.BlockSpec((NUM_CLASSES, EMBED_DIM), lambda i: (0, 0)),
        ],
        out_specs=pl.BlockSpec((_BM, NUM_CLASSES), lambda i: (i, 0)),
        out_shape=jax.ShapeDtypeStruct((BATCH, NUM_CLASSES), jnp.float32),
    )(gathered, weight)


def kernel(nodes, table, weight):
    idx = nodes.astype(jnp.int32).reshape(NW, N_CHUNKS, CHUNK)
    gathered = _sc_gather(table, idx)
    return _tc_matmul(gathered, weight)
